# baseline (speedup 1.0000x reference)
import functools

import jax
import jax.numpy as jnp
from jax import lax
from jax.experimental import pallas as pl
from jax.experimental.pallas import tpu as pltpu
from jax.experimental.pallas import tpu_sc as plsc

_VOCAB = 1000000
_D = 64
_B = 16384
_NNEG = 5
_NW = 32
_CHUNK = 128
_BW = _B // _NW
_NCH = _BW // _CHUNK
_NZW = _B * _NNEG // _NW
_NZG = _NZW // _BW

_mesh = plsc.VectorSubcoreMesh(core_axis_name="c", subcore_axis_name="s")


@functools.partial(
    pl.kernel,
    mesh=_mesh,
    compiler_params=pltpu.CompilerParams(use_tc_tiling_on_sc=False),
    out_type=(
        jax.ShapeDtypeStruct((_B, _D), jnp.float32),
        jax.ShapeDtypeStruct((_B, _D), jnp.float32),
        jax.ShapeDtypeStruct((_B * _NNEG, _D), jnp.float32),
    ),
    scratch_types=[
        pltpu.VMEM((_NCH, _CHUNK), jnp.int32),
        pltpu.VMEM((_NCH, _CHUNK), jnp.int32),
        pltpu.VMEM((_NZW // _CHUNK, _CHUNK), jnp.int32),
        pltpu.VMEM((_BW, _D), jnp.float32),
        pltpu.VMEM((_BW, _D), jnp.float32),
        pltpu.SemaphoreType.DMA,
        pltpu.SemaphoreType.DMA,
    ],
)
def _sgns(in_emb, out_emb, center, context, noise,
          o_center, o_context, o_noise,
          idx_c, idx_x, idx_n, rows_a, rows_b, sem_g, sem_o):
    wid = lax.axis_index("s") * 2 + lax.axis_index("c")
    rows = (rows_a, rows_b)

    pltpu.sync_copy(center.at[wid], idx_c)
    pltpu.sync_copy(context.at[wid], idx_x)
    pltpu.sync_copy(noise.at[wid], idx_n)

    def gather_group(table, idx, first_chunk, buf):
        cps = []
        for j in range(_NCH):
            cps.append(pltpu.async_copy(
                table.at[idx.at[first_chunk + j]],
                buf.at[pl.ds(j * _CHUNK, _CHUNK)],
                sem_g,
            ))
        for cp in cps:
            cp.wait()

    gather_group(in_emb, idx_c, 0, rows[0])
    pltpu.sync_copy(rows[0], o_center.at[pl.ds(wid * _BW, _BW)])

    gather_group(out_emb, idx_x, 0, rows[1])
    pltpu.sync_copy(rows[1], o_context.at[pl.ds(wid * _BW, _BW)])

    for g in range(_NZG):
        buf = rows[g % 2]
        gather_group(out_emb, idx_n, g * _NCH, buf)
        pltpu.sync_copy(buf, o_noise.at[pl.ds(wid * _NZW + g * _BW, _BW)])


def kernel(input_embeddings, output_embeddings, center_words, context_words, noise_words):
    center2d = center_words.astype(jnp.int32).reshape(_NW, _NCH, _CHUNK)
    context2d = context_words.astype(jnp.int32).reshape(_NW, _NCH, _CHUNK)
    noise2d = noise_words.astype(jnp.int32).reshape(_NW, _NZW // _CHUNK, _CHUNK)
    o_center, o_context, o_noise = _sgns(
        input_embeddings, output_embeddings, center2d, context2d, noise2d)
    return (o_center, o_context, o_noise.reshape(_B, _NNEG, _D))


# speedup vs baseline: 1.0022x; 1.0022x over previous
import functools

import jax
import jax.numpy as jnp
from jax import lax
from jax.experimental import pallas as pl
from jax.experimental.pallas import tpu as pltpu
from jax.experimental.pallas import tpu_sc as plsc

_VOCAB = 1000000
_D = 64
_B = 16384
_NNEG = 5
_NW = 32
_G = 512
_BW = _B // _NW
_NZW = _B * _NNEG // _NW
_NG = (2 * _BW + _NZW) // _G
_NSLOT = 3

_mesh = plsc.VectorSubcoreMesh(core_axis_name="c", subcore_axis_name="s")


@functools.partial(
    pl.kernel,
    mesh=_mesh,
    compiler_params=pltpu.CompilerParams(use_tc_tiling_on_sc=False),
    out_type=(
        jax.ShapeDtypeStruct((_B, _D), jnp.float32),
        jax.ShapeDtypeStruct((_B, _D), jnp.float32),
        jax.ShapeDtypeStruct((_B * _NNEG, _D), jnp.float32),
    ),
    scratch_types=[
        pltpu.VMEM((_NG * _G,), jnp.int32),
        pltpu.VMEM((_G, _D), jnp.float32),
        pltpu.VMEM((_G, _D), jnp.float32),
        pltpu.VMEM((_G, _D), jnp.float32),
        pltpu.SemaphoreType.DMA,
        pltpu.SemaphoreType.DMA,
        pltpu.SemaphoreType.DMA,
        pltpu.SemaphoreType.DMA,
    ],
)
def _sgns(in_emb, out_emb, center, context, noise,
          o_center, o_context, o_noise,
          idx, buf0, buf1, buf2, sem_g, sem_o0, sem_o1, sem_o2):
    wid = lax.axis_index("s") * 2 + lax.axis_index("c")
    bufs = (buf0, buf1, buf2)
    sems_o = (sem_o0, sem_o1, sem_o2)

    pltpu.sync_copy(center.at[pl.ds(wid * _BW, _BW)], idx.at[pl.ds(0, _BW)])
    pltpu.sync_copy(context.at[pl.ds(wid * _BW, _BW)], idx.at[pl.ds(_BW, _BW)])
    pltpu.sync_copy(noise.at[pl.ds(wid * _NZW, _NZW)], idx.at[pl.ds(2 * _BW, _NZW)])

    tasks = (
        [(in_emb, o_center, wid * _BW)]
        + [(out_emb, o_context, wid * _BW)]
        + [(out_emb, o_noise, wid * _NZW + k * _G) for k in range(_NZW // _G)]
    )

    def fire_gather(g):
        table = tasks[g][0]
        return pltpu.async_copy(
            table.at[idx.at[pl.ds(g * _G, _G)]], bufs[g % _NSLOT], sem_g)

    def fire_out(g):
        _, out, base = tasks[g]
        return pltpu.async_copy(
            bufs[g % _NSLOT], out.at[pl.ds(base, _G)], sems_o[g % _NSLOT])

    gh = [None] * _NG
    oh = [None] * _NG
    for g in range(_NG):
        if g >= _NSLOT:
            oh[g - _NSLOT].wait()
        gh[g] = fire_gather(g)
        if g >= 1:
            gh[g - 1].wait()
            oh[g - 1] = fire_out(g - 1)
    gh[_NG - 1].wait()
    oh[_NG - 1] = fire_out(_NG - 1)
    for g in range(_NG - _NSLOT, _NG):
        oh[g].wait()


def kernel(input_embeddings, output_embeddings, center_words, context_words, noise_words):
    center1d = center_words.astype(jnp.int32)
    context1d = context_words.astype(jnp.int32)
    noise1d = noise_words.astype(jnp.int32).reshape(_B * _NNEG)
    o_center, o_context, o_noise = _sgns(
        input_embeddings, output_embeddings, center1d, context1d, noise1d)
    return (o_center, o_context, o_noise.reshape(_B, _NNEG, _D))
